# baseline (speedup 1.0000x reference)
import functools

import jax
import jax.numpy as jnp
from jax import lax
from jax.experimental import pallas as pl
from jax.experimental.pallas import tpu as pltpu
from jax.experimental.pallas import tpu_sc as plsc

TOKENS_PER_BLOCK = 4096
N_GATES = 64
TOKENS = 32768
NUM_WORKERS = 32
TOK_PER_WORKER = TOKENS // NUM_WORKERS
GROUPS = TOK_PER_WORKER // 16


def _gate_matmul_kernel(inp_ref, w_ref, b_ref, gt_ref):
    x = inp_ref[...]
    w = w_ref[...]
    gt = lax.dot_general(w, x, (((1,), (1,)), ((), ())),
                         preferred_element_type=jnp.float32)
    gt = gt + b_ref[...][:, 0:1]
    for k in range(TOKENS_PER_BLOCK // TOK_PER_WORKER):
        gt_ref[k] = gt[:, k * TOK_PER_WORKER:(k + 1) * TOK_PER_WORKER]


def _sc_topk_body(gate_hbm, i1_hbm, i2_hbm, s1_hbm, s2_hbm,
                  slab_v, i1_v, i2_v, s1_v, s2_v):
    wid = lax.axis_index("s") * 2 + lax.axis_index("c")
    pltpu.sync_copy(gate_hbm.at[wid], slab_v)

    def group_body(j, carry):
        off = pl.multiple_of(j * 16, 16)
        tok = pl.ds(off, 16)
        m1 = jnp.full((16,), -jnp.inf, jnp.float32)
        m2 = jnp.full((16,), -jnp.inf, jnp.float32)
        i1 = jnp.zeros((16,), jnp.int32)
        i2 = jnp.zeros((16,), jnp.int32)
        for g in range(N_GATES):
            v = slab_v[g, tok]
            gvec = jnp.full((16,), g, jnp.int32)
            b1 = v > m1
            b2 = v > m2
            m2 = jnp.where(b1, m1, jnp.where(b2, v, m2))
            i2 = jnp.where(b1, i1, jnp.where(b2, gvec, i2))
            m1 = jnp.where(b1, v, m1)
            i1 = jnp.where(b1, gvec, i1)
        e2 = jnp.exp(m2 - m1)
        denom = 1.0 + e2
        i1_v[tok] = i1
        i2_v[tok] = i2
        s1_v[tok] = 1.0 / denom
        s2_v[tok] = e2 / denom
        return carry

    lax.fori_loop(0, GROUPS, group_body, 0)
    out = pl.ds(wid * TOK_PER_WORKER, TOK_PER_WORKER)
    pltpu.sync_copy(i1_v, i1_hbm.at[out])
    pltpu.sync_copy(i2_v, i2_hbm.at[out])
    pltpu.sync_copy(s1_v, s1_hbm.at[out])
    pltpu.sync_copy(s2_v, s2_hbm.at[out])


_sc_topk = functools.partial(
    pl.kernel,
    out_type=[
        jax.ShapeDtypeStruct((TOKENS,), jnp.int32),
        jax.ShapeDtypeStruct((TOKENS,), jnp.int32),
        jax.ShapeDtypeStruct((TOKENS,), jnp.float32),
        jax.ShapeDtypeStruct((TOKENS,), jnp.float32),
    ],
    scratch_types=[
        pltpu.VMEM((N_GATES, TOK_PER_WORKER), jnp.float32),
        pltpu.VMEM((TOK_PER_WORKER,), jnp.int32),
        pltpu.VMEM((TOK_PER_WORKER,), jnp.int32),
        pltpu.VMEM((TOK_PER_WORKER,), jnp.float32),
        pltpu.VMEM((TOK_PER_WORKER,), jnp.float32),
    ],
    mesh=plsc.VectorSubcoreMesh(core_axis_name="c", subcore_axis_name="s"),
)(_sc_topk_body)


def kernel(inp, W, b):
    tokens, d_model = inp.shape
    n_gates = W.shape[0]
    b2 = jnp.broadcast_to(b.reshape(n_gates, 1), (n_gates, 128))
    grid = (tokens // TOKENS_PER_BLOCK,)
    chunks_per_block = TOKENS_PER_BLOCK // TOK_PER_WORKER
    gate_t = pl.pallas_call(
        _gate_matmul_kernel,
        grid=grid,
        in_specs=[
            pl.BlockSpec((TOKENS_PER_BLOCK, d_model), lambda i: (i, 0)),
            pl.BlockSpec((n_gates, d_model), lambda i: (0, 0)),
            pl.BlockSpec((n_gates, 128), lambda i: (0, 0)),
        ],
        out_specs=pl.BlockSpec(
            (chunks_per_block, n_gates, TOK_PER_WORKER), lambda i: (i, 0, 0)),
        out_shape=jax.ShapeDtypeStruct(
            (NUM_WORKERS, n_gates, TOK_PER_WORKER), jnp.float32),
        compiler_params=pltpu.CompilerParams(
            dimension_semantics=("parallel",)),
    )(inp, W, b2)
    i1, i2, s1, s2 = _sc_topk(gate_t)
    idx = jnp.concatenate([i1[:, None], i2[:, None]], axis=1).reshape(-1)
    score = jnp.concatenate([s1[:, None], s2[:, None]], axis=1)
    return (idx, score[:, None, :])


# speedup vs baseline: 1.5570x; 1.5570x over previous
import jax
import jax.numpy as jnp
from jax import lax
from jax.experimental import pallas as pl
from jax.experimental.pallas import tpu as pltpu

TOKENS_PER_BLOCK = 4096
N_GATES = 64


def _gate_topk_kernel(inp_ref, w_ref, b_ref, idx_ref, score_ref):
    x = inp_ref[...]
    w = w_ref[...]
    gt = lax.dot_general(w, x, (((1,), (1,)), ((), ())),
                         preferred_element_type=jnp.float32)
    gt = gt + b_ref[...][:, 0:1]
    rows = lax.broadcasted_iota(jnp.int32, gt.shape, 0)
    m1 = jnp.max(gt, axis=0, keepdims=True)
    i1 = jnp.min(jnp.where(gt == m1, rows, N_GATES), axis=0, keepdims=True)
    gt2 = jnp.where(rows == i1, -jnp.inf, gt)
    m2 = jnp.max(gt2, axis=0, keepdims=True)
    i2 = jnp.min(jnp.where(gt2 == m2, rows, N_GATES), axis=0, keepdims=True)
    idx_ref[...] = jnp.concatenate([i1, i2], axis=0)
    e2 = jnp.exp(m2 - m1)
    denom = 1.0 + e2
    score_ref[...] = jnp.concatenate([1.0 / denom, e2 / denom], axis=0)


def kernel(inp, W, b):
    tokens, d_model = inp.shape
    n_gates = W.shape[0]
    b2 = jnp.broadcast_to(b.reshape(n_gates, 1), (n_gates, 128))
    grid = (tokens // TOKENS_PER_BLOCK,)
    idx_t, score_t = pl.pallas_call(
        _gate_topk_kernel,
        grid=grid,
        in_specs=[
            pl.BlockSpec((TOKENS_PER_BLOCK, d_model), lambda i: (i, 0)),
            pl.BlockSpec((n_gates, d_model), lambda i: (0, 0)),
            pl.BlockSpec((n_gates, 128), lambda i: (0, 0)),
        ],
        out_specs=[
            pl.BlockSpec((2, TOKENS_PER_BLOCK), lambda i: (0, i)),
            pl.BlockSpec((2, TOKENS_PER_BLOCK), lambda i: (0, i)),
        ],
        out_shape=[
            jax.ShapeDtypeStruct((2, tokens), jnp.int32),
            jax.ShapeDtypeStruct((2, tokens), jnp.float32),
        ],
        compiler_params=pltpu.CompilerParams(
            dimension_semantics=("parallel",)),
    )(inp, W, b2)
    return (idx_t.T.reshape(-1), score_t.T[:, None, :])
